# baseline (speedup 1.0000x reference)
import functools

import jax
import jax.numpy as jnp
from jax import lax
from jax.experimental import pallas as pl
from jax.experimental.pallas import tpu as pltpu
from jax.experimental.pallas import tpu_sc as plsc

N = 10000
NP = 10240
HID = 128
HEADS = 2
G = 64
NTILES = 16
CH = 128
ROWS_PT = NP // NTILES


def _lrelu(v):
    return jnp.where(v >= 0, v, 0.2 * v)


def _attn_prep(x, wg, att_s, att_d, h_ref, as_ref, ad_ref, amax_ref):
    h = jnp.dot(x, wg, preferred_element_type=jnp.float32)
    amaxes = []
    for k in range(HEADS):
        hk = h[:, k * HID:(k + 1) * HID]
        a_s = jnp.sum(hk * att_s[k][None, :], axis=-1)
        a_d = jnp.sum(hk * att_d[k][None, :], axis=-1)
        h_ref[pl.ds(k * NP, N), :] = hk
        as_ref[pl.ds(k * NP, N)] = a_s
        ad_ref[pl.ds(k * NP, N)] = a_d
        amaxes.append(jnp.full((16,), jnp.max(a_s), jnp.float32))
    amax_ref[...] = jnp.stack(amaxes, axis=0)


def _enc_body(meta_ref, wave_ref, wm_ref, bm_ref, ww_ref, bw_ref,
              wc1_ref, wc2_ref, bc_ref, g_ref, b_ref,
              wg_ref, atts_ref, attd_ref,
              h_ref, as_ref, ad_ref, amax_ref):
    m = jax.nn.relu(jnp.dot(meta_ref[...], wm_ref[...],
                            preferred_element_type=jnp.float32) + bm_ref[...])
    wv = jax.nn.relu(jnp.dot(wave_ref[...], ww_ref[...],
                             preferred_element_type=jnp.float32) + bw_ref[...])
    z = (jnp.dot(m, wc1_ref[...], preferred_element_type=jnp.float32)
         + jnp.dot(wv, wc2_ref[...], preferred_element_type=jnp.float32)
         + bc_ref[...])
    mu = jnp.mean(z, axis=0)
    va = jnp.mean(z * z, axis=0) - mu * mu
    x = jax.nn.relu((z - mu) * lax.rsqrt(va + 1e-5) * g_ref[...] + b_ref[...])
    _attn_prep(x, wg_ref[...], atts_ref[...], attd_ref[...],
               h_ref, as_ref, ad_ref, amax_ref)


def _norm_from_edges(acc, den, bg, g, b, xres):
    ys = []
    for k in range(HEADS):
        a = acc[k * NP:k * NP + N]
        dd = den[k * NP:k * NP + N]
        ys.append(a / (dd[:, None] + 1e-16))
    y = jnp.concatenate(ys, axis=1) + bg
    mu = jnp.mean(y, axis=0)
    va = jnp.mean(y * y, axis=0) - mu * mu
    y = (y - mu) * lax.rsqrt(va + 1e-5) * g + b
    if xres is not None:
        y = y + xres
    return jax.nn.relu(y)


def _post_body(has_res, *refs):
    if has_res:
        (acc_ref, den_ref, bg_ref, g_ref, b_ref, xres_ref,
         wg_ref, atts_ref, attd_ref,
         xnew_ref, h_ref, as_ref, ad_ref, amax_ref) = refs
        xres = xres_ref[...]
    else:
        (acc_ref, den_ref, bg_ref, g_ref, b_ref,
         wg_ref, atts_ref, attd_ref,
         xnew_ref, h_ref, as_ref, ad_ref, amax_ref) = refs
        xres = None
    x = _norm_from_edges(acc_ref[...], den_ref[...], bg_ref[...],
                         g_ref[...], b_ref[...], xres)
    xnew_ref[...] = x
    _attn_prep(x, wg_ref[...], atts_ref[...], attd_ref[...],
               h_ref, as_ref, ad_ref, amax_ref)


def _final_body(acc_ref, den_ref, bg_ref, g_ref, b_ref, xres_ref, batch_ref,
                wl1_ref, bl1_ref, wl2_ref, bl2_ref,
                wo1_ref, bo1_ref, wo2_ref, bo2_ref,
                lat_ref, lon_ref):
    x = _norm_from_edges(acc_ref[...], den_ref[...], bg_ref[...],
                         g_ref[...], b_ref[...], xres_ref[...])
    gids = lax.broadcasted_iota(jnp.int32, (G, N), 0)
    oh = (batch_ref[...][None, :] == gids).astype(jnp.float32)
    cnt = jnp.sum(oh, axis=1)
    xs = jnp.dot(oh, x, preferred_element_type=jnp.float32)
    xg = xs / jnp.maximum(cnt, 1.0)[:, None]
    lat = jnp.dot(jax.nn.relu(
        jnp.dot(xg, wl1_ref[...], preferred_element_type=jnp.float32)
        + bl1_ref[...]), wl2_ref[...],
        preferred_element_type=jnp.float32) + bl2_ref[...]
    lon = jnp.dot(jax.nn.relu(
        jnp.dot(xg, wo1_ref[...], preferred_element_type=jnp.float32)
        + bo1_ref[...]), wo2_ref[...],
        preferred_element_type=jnp.float32) + bo2_ref[...]
    lat_ref[...] = lat
    lon_ref[...] = lon


_F32 = jnp.float32
_TC_PARAMS = pltpu.CompilerParams(vmem_limit_bytes=100 * 1024 * 1024)
_ATTN_OUT = (
    jax.ShapeDtypeStruct((2 * NP, HID), _F32),
    jax.ShapeDtypeStruct((2 * NP,), _F32),
    jax.ShapeDtypeStruct((2 * NP,), _F32),
    jax.ShapeDtypeStruct((2, 16), _F32),
)

_enc_call = pl.pallas_call(_enc_body, out_shape=_ATTN_OUT,
                           compiler_params=_TC_PARAMS)
_post_call_nores = pl.pallas_call(
    functools.partial(_post_body, False),
    out_shape=(jax.ShapeDtypeStruct((N, 2 * HID), _F32),) + _ATTN_OUT,
    compiler_params=_TC_PARAMS)
_post_call_res = pl.pallas_call(
    functools.partial(_post_body, True),
    out_shape=(jax.ShapeDtypeStruct((N, 2 * HID), _F32),) + _ATTN_OUT,
    compiler_params=_TC_PARAMS)
_final_call = pl.pallas_call(
    _final_body,
    out_shape=(jax.ShapeDtypeStruct((G, 1), _F32),
               jax.ShapeDtypeStruct((G, 1), _F32)),
    compiler_params=_TC_PARAMS)


def _edge_body(epad, src_hbm, dst_hbm, as_hbm, ad_hbm, amax_hbm, h_hbm,
               acc_hbm, den_hbm,
               srcA, dstA, soffA, doffA, asgA, adgA, exvA, hrowA,
               srcB, dstB, soffB, doffB, asgB, adgB, exvB, hrowB, amaxv,
               acc_sh, den_sh, semA, semB):
    k = lax.axis_index("c")
    s = lax.axis_index("s")
    koff = k * NP
    chunks = epad // (NTILES * CH)

    pltpu.sync_copy(amax_hbm.at[k], amaxv)
    amax_vec = amaxv[...]

    def _zrow(i, _):
        for f in range(HID // 16):
            hrowA[i, pl.ds(f * 16, 16)] = jnp.zeros((16,), _F32)
        return 0
    lax.fori_loop(0, CH, _zrow, 0)
    for j in range(CH // 16):
        exvA[pl.ds(j * 16, 16)] = jnp.zeros((16,), _F32)
    for j in range(ROWS_PT // CH):
        r0 = s * ROWS_PT + j * CH
        pltpu.sync_copy(hrowA, acc_sh.at[pl.ds(r0, CH)])
        pltpu.sync_copy(exvA, den_sh.at[pl.ds(r0, CH)])
    plsc.subcore_barrier()

    bufs = ((srcA, dstA, soffA, doffA, asgA, adgA, exvA, hrowA, semA),
            (srcB, dstB, soffB, doffB, asgB, adgB, exvB, hrowB, semB))

    def _issue(i, buf):
        srcv, dstv, soffv, doffv, asg, adg, exv, hrow, sem = buf
        row = s * chunks + i
        pltpu.sync_copy(src_hbm.at[row], srcv)
        pltpu.sync_copy(dst_hbm.at[row], dstv)
        for j in range(CH // 16):
            sl = pl.ds(j * 16, 16)
            soffv[sl] = srcv[sl] + koff
            doffv[sl] = dstv[sl] + koff
        pltpu.async_copy(as_hbm.at[soffv], asg, sem)
        pltpu.async_copy(ad_hbm.at[doffv], adg, sem)
        pltpu.async_copy(h_hbm.at[soffv], hrow, sem)

    def _wait(i, buf):
        srcv, dstv, soffv, doffv, asg, adg, exv, hrow, sem = buf
        pltpu.make_async_copy(as_hbm.at[soffv], asg, sem).wait()
        pltpu.make_async_copy(ad_hbm.at[doffv], adg, sem).wait()
        pltpu.make_async_copy(h_hbm.at[soffv], hrow, sem).wait()

    def _process(i, buf):
        srcv, dstv, soffv, doffv, asg, adg, exv, hrow, sem = buf
        for j in range(CH // 16):
            sl = pl.ds(j * 16, 16)
            av = asg[sl]
            dv = adg[sl]
            al = _lrelu(av + dv)
            cc = _lrelu(amax_vec + dv)
            exv[sl] = jnp.exp(al - cc)

        def erow(e, _):
            spl = plsc.load_gather(exv, [jnp.full((16,), e, jnp.int32)])
            for f in range(HID // 16):
                fsl = pl.ds(f * 16, 16)
                hrow[e, fsl] = hrow[e, fsl] * spl
            return 0
        lax.fori_loop(0, CH, erow, 0)

        pltpu.sync_copy(exv, den_sh.at[dstv], add=True)
        pltpu.sync_copy(hrow, acc_sh.at[dstv], add=True)

    _issue(0, bufs[0])

    def body(jj, _):
        for b in (0, 1):
            i = 2 * jj + b
            nxt = i + 1

            @pl.when(nxt < chunks)
            def _():
                _issue(nxt, bufs[1 - b])
            _wait(i, bufs[b])
            _process(i, bufs[b])
        return 0
    lax.fori_loop(0, chunks // 2, body, 0)

    plsc.subcore_barrier()
    for j in range(ROWS_PT // CH):
        r0 = s * ROWS_PT + j * CH
        pltpu.sync_copy(acc_sh.at[pl.ds(r0, CH)], hrowA)
        pltpu.sync_copy(hrowA, acc_hbm.at[pl.ds(koff + r0, CH)])
        pltpu.sync_copy(den_sh.at[pl.ds(r0, CH)], exvA)
        pltpu.sync_copy(exvA, den_hbm.at[pl.ds(koff + r0, CH)])


def _make_edge_call(epad):
    mesh = plsc.VectorSubcoreMesh(core_axis_name="c", subcore_axis_name="s")
    nrows = epad // CH
    chunks = epad // (NTILES * CH)
    return pl.kernel(
        functools.partial(_edge_body, epad),
        out_type=(jax.ShapeDtypeStruct((2 * NP, HID), _F32),
                  jax.ShapeDtypeStruct((2 * NP,), _F32)),
        mesh=mesh,
        compiler_params=pltpu.CompilerParams(needs_layout_passes=False),
        scratch_types=(
            [pltpu.VMEM((CH,), jnp.int32)] * 4 +
            [pltpu.VMEM((CH,), _F32)] * 3 +
            [pltpu.VMEM((CH, HID), _F32)] +
            [pltpu.VMEM((CH,), jnp.int32)] * 4 +
            [pltpu.VMEM((CH,), _F32)] * 3 +
            [pltpu.VMEM((CH, HID), _F32)] +
            [pltpu.VMEM((16,), _F32)] +
            [pltpu.VMEM_SHARED((NP, HID), _F32),
             pltpu.VMEM_SHARED((NP,), _F32),
             pltpu.SemaphoreType.DMA,
             pltpu.SemaphoreType.DMA]
        ),
    )


def kernel(metadata, waveform_features, edge_index, batch,
           W_meta, b_meta, W_wave, b_wave, W_comb, b_comb, bnc_g, bnc_b,
           Wg0, as0, ad0, bg0, bn0_g, bn0_b,
           Wg1, as1, ad1, bg1, bn1_g, bn1_b,
           Wg2, as2, ad2, bg2, bn2_g, bn2_b,
           Wl1, bl1, Wl2, bl2, Wo1, bo1, Wo2, bo2):
    loop = jnp.arange(N, dtype=edge_index.dtype)
    src = jnp.concatenate([edge_index[0], loop])
    dst = jnp.concatenate([edge_index[1], loop])
    e2 = src.shape[0]
    quantum = NTILES * CH * 8
    epad = ((e2 + quantum - 1) // quantum) * quantum
    pad = jnp.full((epad - e2,), NP - 1, dtype=src.dtype)
    src_p = jnp.concatenate([src, pad]).reshape(epad // CH, CH)
    dst_p = jnp.concatenate([dst, pad]).reshape(epad // CH, CH)

    edge_call = _make_edge_call(epad)

    h, asf, adf, amax = _enc_call(
        metadata, waveform_features, W_meta, b_meta, W_wave, b_wave,
        W_comb[:HID], W_comb[HID:], b_comb, bnc_g, bnc_b, Wg0, as0, ad0)

    acc0, den0 = edge_call(src_p, dst_p, asf, adf, amax, h)
    x1, h, asf, adf, amax = _post_call_nores(
        acc0, den0, bg0, bn0_g, bn0_b, Wg1, as1, ad1)

    acc1, den1 = edge_call(src_p, dst_p, asf, adf, amax, h)
    x2, h, asf, adf, amax = _post_call_res(
        acc1, den1, bg1, bn1_g, bn1_b, x1, Wg2, as2, ad2)

    acc2, den2 = edge_call(src_p, dst_p, asf, adf, amax, h)
    lat, lon = _final_call(
        acc2, den2, bg2, bn2_g, bn2_b, x2, batch,
        Wl1, bl1, Wl2, bl2, Wo1, bo1, Wo2, bo2)
    return lat, lon


# speedup vs baseline: 1.3981x; 1.3981x over previous
import functools

import jax
import jax.numpy as jnp
from jax import lax
from jax.experimental import pallas as pl
from jax.experimental.pallas import tpu as pltpu
from jax.experimental.pallas import tpu_sc as plsc

N = 10000
NP = 10240
HID = 128
HEADS = 2
G = 64
NTILES = 16
CH = 128
ROWS_PT = NP // NTILES


def _lrelu(v):
    return jnp.where(v >= 0, v, 0.2 * v)


def _attn_prep(x, wg, att_s, att_d, h_ref, as_ref, ad_ref, amax_ref):
    h = jnp.dot(x, wg, preferred_element_type=jnp.float32)
    amaxes = []
    for k in range(HEADS):
        hk = h[:, k * HID:(k + 1) * HID]
        a_s = jnp.sum(hk * att_s[k][None, :], axis=-1)
        a_d = jnp.sum(hk * att_d[k][None, :], axis=-1)
        h_ref[pl.ds(k * NP, N), :] = hk
        as_ref[pl.ds(k * NP, N)] = a_s
        ad_ref[pl.ds(k * NP, N)] = a_d
        amaxes.append(jnp.full((16,), jnp.max(a_s), jnp.float32))
    amax_ref[...] = jnp.stack(amaxes, axis=0)


def _enc_body(meta_ref, wave_ref, wm_ref, bm_ref, ww_ref, bw_ref,
              wc1_ref, wc2_ref, bc_ref, g_ref, b_ref,
              wg_ref, atts_ref, attd_ref,
              h_ref, as_ref, ad_ref, amax_ref):
    m = jax.nn.relu(jnp.dot(meta_ref[...], wm_ref[...],
                            preferred_element_type=jnp.float32) + bm_ref[...])
    wv = jax.nn.relu(jnp.dot(wave_ref[...], ww_ref[...],
                             preferred_element_type=jnp.float32) + bw_ref[...])
    z = (jnp.dot(m, wc1_ref[...], preferred_element_type=jnp.float32)
         + jnp.dot(wv, wc2_ref[...], preferred_element_type=jnp.float32)
         + bc_ref[...])
    mu = jnp.mean(z, axis=0)
    va = jnp.mean(z * z, axis=0) - mu * mu
    x = jax.nn.relu((z - mu) * lax.rsqrt(va + 1e-5) * g_ref[...] + b_ref[...])
    _attn_prep(x, wg_ref[...], atts_ref[...], attd_ref[...],
               h_ref, as_ref, ad_ref, amax_ref)


def _norm_from_edges(acc, den, bg, g, b, xres):
    ys = []
    for k in range(HEADS):
        a = acc[k * NP:k * NP + N]
        dd = den[k * NP:k * NP + N]
        ys.append(a / (dd[:, None] + 1e-16))
    y = jnp.concatenate(ys, axis=1) + bg
    mu = jnp.mean(y, axis=0)
    va = jnp.mean(y * y, axis=0) - mu * mu
    y = (y - mu) * lax.rsqrt(va + 1e-5) * g + b
    if xres is not None:
        y = y + xres
    return jax.nn.relu(y)


def _post_body(has_res, *refs):
    if has_res:
        (acc_ref, den_ref, bg_ref, g_ref, b_ref, xres_ref,
         wg_ref, atts_ref, attd_ref,
         xnew_ref, h_ref, as_ref, ad_ref, amax_ref) = refs
        xres = xres_ref[...]
    else:
        (acc_ref, den_ref, bg_ref, g_ref, b_ref,
         wg_ref, atts_ref, attd_ref,
         xnew_ref, h_ref, as_ref, ad_ref, amax_ref) = refs
        xres = None
    x = _norm_from_edges(acc_ref[...], den_ref[...], bg_ref[...],
                         g_ref[...], b_ref[...], xres)
    xnew_ref[...] = x
    _attn_prep(x, wg_ref[...], atts_ref[...], attd_ref[...],
               h_ref, as_ref, ad_ref, amax_ref)


def _final_body(acc_ref, den_ref, bg_ref, g_ref, b_ref, xres_ref, batch_ref,
                wl1_ref, bl1_ref, wl2_ref, bl2_ref,
                wo1_ref, bo1_ref, wo2_ref, bo2_ref,
                lat_ref, lon_ref):
    x = _norm_from_edges(acc_ref[...], den_ref[...], bg_ref[...],
                         g_ref[...], b_ref[...], xres_ref[...])
    gids = lax.broadcasted_iota(jnp.int32, (G, N), 0)
    oh = (batch_ref[...][None, :] == gids).astype(jnp.float32)
    cnt = jnp.sum(oh, axis=1)
    xs = jnp.dot(oh, x, preferred_element_type=jnp.float32)
    xg = xs / jnp.maximum(cnt, 1.0)[:, None]
    lat = jnp.dot(jax.nn.relu(
        jnp.dot(xg, wl1_ref[...], preferred_element_type=jnp.float32)
        + bl1_ref[...]), wl2_ref[...],
        preferred_element_type=jnp.float32) + bl2_ref[...]
    lon = jnp.dot(jax.nn.relu(
        jnp.dot(xg, wo1_ref[...], preferred_element_type=jnp.float32)
        + bo1_ref[...]), wo2_ref[...],
        preferred_element_type=jnp.float32) + bo2_ref[...]
    lat_ref[...] = lat
    lon_ref[...] = lon


_F32 = jnp.float32
_TC_PARAMS = pltpu.CompilerParams(vmem_limit_bytes=100 * 1024 * 1024)
_ATTN_OUT = (
    jax.ShapeDtypeStruct((2 * NP, HID), _F32),
    jax.ShapeDtypeStruct((2 * NP,), _F32),
    jax.ShapeDtypeStruct((2 * NP,), _F32),
    jax.ShapeDtypeStruct((2, 16), _F32),
)

_enc_call = pl.pallas_call(_enc_body, out_shape=_ATTN_OUT,
                           compiler_params=_TC_PARAMS)
_post_call_nores = pl.pallas_call(
    functools.partial(_post_body, False),
    out_shape=(jax.ShapeDtypeStruct((N, 2 * HID), _F32),) + _ATTN_OUT,
    compiler_params=_TC_PARAMS)
_post_call_res = pl.pallas_call(
    functools.partial(_post_body, True),
    out_shape=(jax.ShapeDtypeStruct((N, 2 * HID), _F32),) + _ATTN_OUT,
    compiler_params=_TC_PARAMS)
_final_call = pl.pallas_call(
    _final_body,
    out_shape=(jax.ShapeDtypeStruct((G, 1), _F32),
               jax.ShapeDtypeStruct((G, 1), _F32)),
    compiler_params=_TC_PARAMS)


def _edge_body(epad, src_hbm, dst_hbm, as_hbm, ad_hbm, amax_hbm, h_hbm,
               acc_hbm, den_hbm,
               srcv, dstv, soffv, doffv, asg, adg, exv, hrow, amaxv,
               acc_sh, den_sh, sem, sem2):
    k = lax.axis_index("c")
    s = lax.axis_index("s")
    koff = k * NP
    chunks = epad // (NTILES * CH)
    ept = chunks * CH

    pltpu.sync_copy(amax_hbm.at[k], amaxv)
    amax_vec = amaxv[...]

    def _zrow(i, _):
        for f in range(HID // 16):
            hrow[i, pl.ds(f * 16, 16)] = jnp.zeros((16,), _F32)
        return 0
    lax.fori_loop(0, CH, _zrow, 0)
    for j in range(CH // 16):
        exv[pl.ds(j * 16, 16)] = jnp.zeros((16,), _F32)
    for j in range(ROWS_PT // CH):
        r0 = s * ROWS_PT + j * CH
        pltpu.sync_copy(hrow, acc_sh.at[pl.ds(r0, CH)])
        pltpu.sync_copy(exv, den_sh.at[pl.ds(r0, CH)])
    plsc.subcore_barrier()

    def chunk(i, _):
        base = s * ept + i * CH
        pltpu.sync_copy(src_hbm.at[pl.ds(base, CH)], srcv)
        pltpu.sync_copy(dst_hbm.at[pl.ds(base, CH)], dstv)
        for j in range(CH // 16):
            sl = pl.ds(j * 16, 16)
            soffv[sl] = srcv[sl] + koff
            doffv[sl] = dstv[sl] + koff
        cp1 = pltpu.async_copy(as_hbm.at[soffv], asg, sem)
        cp2 = pltpu.async_copy(ad_hbm.at[doffv], adg, sem)
        cp3 = pltpu.async_copy(h_hbm.at[soffv], hrow, sem2)
        cp1.wait()
        cp2.wait()
        for j in range(CH // 16):
            sl = pl.ds(j * 16, 16)
            av = asg[sl]
            dv = adg[sl]
            al = _lrelu(av + dv)
            cc = _lrelu(amax_vec + dv)
            exv[sl] = jnp.exp(al - cc)
        cp3.wait()

        def erow(e):
            spl = plsc.load_gather(exv, [jnp.full((16,), e, jnp.int32)])
            for f in range(HID // 16):
                fsl = pl.ds(f * 16, 16)
                hrow[e, fsl] = hrow[e, fsl] * spl
        plsc.parallel_loop(0, CH, 1, unroll=4)(erow)

        pltpu.sync_copy(exv, den_sh.at[dstv], add=True)
        pltpu.sync_copy(hrow, acc_sh.at[dstv], add=True)
        return 0
    lax.fori_loop(0, chunks, chunk, 0)

    plsc.subcore_barrier()
    for j in range(ROWS_PT // CH):
        r0 = s * ROWS_PT + j * CH
        pltpu.sync_copy(acc_sh.at[pl.ds(r0, CH)], hrow)
        pltpu.sync_copy(hrow, acc_hbm.at[pl.ds(koff + r0, CH)])
        pltpu.sync_copy(den_sh.at[pl.ds(r0, CH)], exv)
        pltpu.sync_copy(exv, den_hbm.at[pl.ds(koff + r0, CH)])


def _make_edge_call(epad):
    mesh = plsc.VectorSubcoreMesh(core_axis_name="c", subcore_axis_name="s")
    return pl.kernel(
        functools.partial(_edge_body, epad),
        out_type=(jax.ShapeDtypeStruct((2 * NP, HID), _F32),
                  jax.ShapeDtypeStruct((2 * NP,), _F32)),
        mesh=mesh,
        compiler_params=pltpu.CompilerParams(needs_layout_passes=False),
        scratch_types=(
            [pltpu.VMEM((CH,), jnp.int32)] * 4 +
            [pltpu.VMEM((CH,), _F32)] * 3 +
            [pltpu.VMEM((CH, HID), _F32)] +
            [pltpu.VMEM((16,), _F32)] +
            [pltpu.VMEM_SHARED((NP, HID), _F32),
             pltpu.VMEM_SHARED((NP,), _F32),
             pltpu.SemaphoreType.DMA,
             pltpu.SemaphoreType.DMA]
        ),
    )


def kernel(metadata, waveform_features, edge_index, batch,
           W_meta, b_meta, W_wave, b_wave, W_comb, b_comb, bnc_g, bnc_b,
           Wg0, as0, ad0, bg0, bn0_g, bn0_b,
           Wg1, as1, ad1, bg1, bn1_g, bn1_b,
           Wg2, as2, ad2, bg2, bn2_g, bn2_b,
           Wl1, bl1, Wl2, bl2, Wo1, bo1, Wo2, bo2):
    loop = jnp.arange(N, dtype=edge_index.dtype)
    src = jnp.concatenate([edge_index[0], loop])
    dst = jnp.concatenate([edge_index[1], loop])
    e2 = src.shape[0]
    epad = ((e2 + NTILES * CH - 1) // (NTILES * CH)) * (NTILES * CH)
    pad = jnp.full((epad - e2,), NP - 1, dtype=src.dtype)
    src_p = jnp.concatenate([src, pad])
    dst_p = jnp.concatenate([dst, pad])

    edge_call = _make_edge_call(epad)

    h, asf, adf, amax = _enc_call(
        metadata, waveform_features, W_meta, b_meta, W_wave, b_wave,
        W_comb[:HID], W_comb[HID:], b_comb, bnc_g, bnc_b, Wg0, as0, ad0)

    acc0, den0 = edge_call(src_p, dst_p, asf, adf, amax, h)
    x1, h, asf, adf, amax = _post_call_nores(
        acc0, den0, bg0, bn0_g, bn0_b, Wg1, as1, ad1)

    acc1, den1 = edge_call(src_p, dst_p, asf, adf, amax, h)
    x2, h, asf, adf, amax = _post_call_res(
        acc1, den1, bg1, bn1_g, bn1_b, x1, Wg2, as2, ad2)

    acc2, den2 = edge_call(src_p, dst_p, asf, adf, amax, h)
    lat, lon = _final_call(
        acc2, den2, bg2, bn2_g, bn2_b, x2, batch,
        Wl1, bl1, Wl2, bl2, Wo1, bo1, Wo2, bo2)
    return lat, lon


# speedup vs baseline: 1.9007x; 1.3595x over previous
import functools

import jax
import jax.numpy as jnp
from jax import lax
from jax.experimental import pallas as pl
from jax.experimental.pallas import tpu as pltpu
from jax.experimental.pallas import tpu_sc as plsc

N = 10000
NP = 10240
HID = 128
HEADS = 2
G = 64
NTILES = 16
CH = 128
ROWS_PT = NP // NTILES


def _lrelu(v):
    return jnp.where(v >= 0, v, 0.2 * v)


def _attn_prep(x, wg, att_s, att_d, h_ref, as_ref, ad_ref, amax_ref):
    h = jnp.dot(x, wg, preferred_element_type=jnp.float32)
    amaxes = []
    for k in range(HEADS):
        hk = h[:, k * HID:(k + 1) * HID]
        a_s = jnp.sum(hk * att_s[k][None, :], axis=-1)
        a_d = jnp.sum(hk * att_d[k][None, :], axis=-1)
        h_ref[pl.ds(k * NP, N), :] = hk
        as_ref[pl.ds(k * NP, N)] = a_s
        ad_ref[pl.ds(k * NP, N)] = a_d
        amaxes.append(jnp.full((16,), jnp.max(a_s), jnp.float32))
    amax_ref[...] = jnp.stack(amaxes, axis=0)


def _enc_body(meta_ref, wave_ref, wm_ref, bm_ref, ww_ref, bw_ref,
              wc1_ref, wc2_ref, bc_ref, g_ref, b_ref,
              wg_ref, atts_ref, attd_ref,
              h_ref, as_ref, ad_ref, amax_ref):
    m = jax.nn.relu(jnp.dot(meta_ref[...], wm_ref[...],
                            preferred_element_type=jnp.float32) + bm_ref[...])
    wv = jax.nn.relu(jnp.dot(wave_ref[...], ww_ref[...],
                             preferred_element_type=jnp.float32) + bw_ref[...])
    z = (jnp.dot(m, wc1_ref[...], preferred_element_type=jnp.float32)
         + jnp.dot(wv, wc2_ref[...], preferred_element_type=jnp.float32)
         + bc_ref[...])
    mu = jnp.mean(z, axis=0)
    va = jnp.mean(z * z, axis=0) - mu * mu
    x = jax.nn.relu((z - mu) * lax.rsqrt(va + 1e-5) * g_ref[...] + b_ref[...])
    _attn_prep(x, wg_ref[...], atts_ref[...], attd_ref[...],
               h_ref, as_ref, ad_ref, amax_ref)


def _norm_from_edges(acc, den, bg, g, b, xres):
    ys = []
    for k in range(HEADS):
        a = acc[k * NP:k * NP + N]
        dd = den[k * NP:k * NP + N]
        ys.append(a / (dd[:, None] + 1e-16))
    y = jnp.concatenate(ys, axis=1) + bg
    mu = jnp.mean(y, axis=0)
    va = jnp.mean(y * y, axis=0) - mu * mu
    y = (y - mu) * lax.rsqrt(va + 1e-5) * g + b
    if xres is not None:
        y = y + xres
    return jax.nn.relu(y)


def _post_body(has_res, *refs):
    if has_res:
        (acc_ref, den_ref, bg_ref, g_ref, b_ref, xres_ref,
         wg_ref, atts_ref, attd_ref,
         xnew_ref, h_ref, as_ref, ad_ref, amax_ref) = refs
        xres = xres_ref[...]
    else:
        (acc_ref, den_ref, bg_ref, g_ref, b_ref,
         wg_ref, atts_ref, attd_ref,
         xnew_ref, h_ref, as_ref, ad_ref, amax_ref) = refs
        xres = None
    x = _norm_from_edges(acc_ref[...], den_ref[...], bg_ref[...],
                         g_ref[...], b_ref[...], xres)
    xnew_ref[...] = x
    _attn_prep(x, wg_ref[...], atts_ref[...], attd_ref[...],
               h_ref, as_ref, ad_ref, amax_ref)


def _final_body(acc_ref, den_ref, bg_ref, g_ref, b_ref, xres_ref, batch_ref,
                wl1_ref, bl1_ref, wl2_ref, bl2_ref,
                wo1_ref, bo1_ref, wo2_ref, bo2_ref,
                lat_ref, lon_ref):
    x = _norm_from_edges(acc_ref[...], den_ref[...], bg_ref[...],
                         g_ref[...], b_ref[...], xres_ref[...])
    gids = lax.broadcasted_iota(jnp.int32, (G, N), 0)
    oh = (batch_ref[...][None, :] == gids).astype(jnp.float32)
    cnt = jnp.sum(oh, axis=1)
    xs = jnp.dot(oh, x, preferred_element_type=jnp.float32)
    xg = xs / jnp.maximum(cnt, 1.0)[:, None]
    lat = jnp.dot(jax.nn.relu(
        jnp.dot(xg, wl1_ref[...], preferred_element_type=jnp.float32)
        + bl1_ref[...]), wl2_ref[...],
        preferred_element_type=jnp.float32) + bl2_ref[...]
    lon = jnp.dot(jax.nn.relu(
        jnp.dot(xg, wo1_ref[...], preferred_element_type=jnp.float32)
        + bo1_ref[...]), wo2_ref[...],
        preferred_element_type=jnp.float32) + bo2_ref[...]
    lat_ref[...] = lat
    lon_ref[...] = lon


_F32 = jnp.float32
_TC_PARAMS = pltpu.CompilerParams(vmem_limit_bytes=100 * 1024 * 1024)
_ATTN_OUT = (
    jax.ShapeDtypeStruct((2 * NP, HID), _F32),
    jax.ShapeDtypeStruct((2 * NP,), _F32),
    jax.ShapeDtypeStruct((2 * NP,), _F32),
    jax.ShapeDtypeStruct((2, 16), _F32),
)

_enc_call = pl.pallas_call(_enc_body, out_shape=_ATTN_OUT,
                           compiler_params=_TC_PARAMS)
_post_call_nores = pl.pallas_call(
    functools.partial(_post_body, False),
    out_shape=(jax.ShapeDtypeStruct((N, 2 * HID), _F32),) + _ATTN_OUT,
    compiler_params=_TC_PARAMS)
_post_call_res = pl.pallas_call(
    functools.partial(_post_body, True),
    out_shape=(jax.ShapeDtypeStruct((N, 2 * HID), _F32),) + _ATTN_OUT,
    compiler_params=_TC_PARAMS)
_final_call = pl.pallas_call(
    _final_body,
    out_shape=(jax.ShapeDtypeStruct((G, 1), _F32),
               jax.ShapeDtypeStruct((G, 1), _F32)),
    compiler_params=_TC_PARAMS)


def _edge_body(epad, src_hbm, dst_hbm, as_hbm, ad_hbm, amax_hbm, h_hbm,
               acc_hbm, den_hbm,
               srcA, dstA, soffA, doffA, asgA, adgA, exvA, hrowA,
               srcB, dstB, soffB, doffB, asgB, adgB, exvB, hrowB, amaxv,
               acc_sh, den_sh, semA1, semA2, semA3, semB1, semB2, semB3):
    k = lax.axis_index("c")
    s = lax.axis_index("s")
    koff = k * NP
    chunks = epad // (NTILES * CH)
    ept = chunks * CH

    pltpu.sync_copy(amax_hbm.at[k], amaxv)
    amax_vec = amaxv[...]

    def _zrow(i, _):
        for f in range(HID // 16):
            hrowA[i, pl.ds(f * 16, 16)] = jnp.zeros((16,), _F32)
        return 0
    lax.fori_loop(0, CH, _zrow, 0)
    for j in range(CH // 16):
        exvA[pl.ds(j * 16, 16)] = jnp.zeros((16,), _F32)
    for j in range(ROWS_PT // CH):
        r0 = s * ROWS_PT + j * CH
        pltpu.sync_copy(hrowA, acc_sh.at[pl.ds(r0, CH)])
        pltpu.sync_copy(exvA, den_sh.at[pl.ds(r0, CH)])
    plsc.subcore_barrier()

    setA = (srcA, dstA, soffA, doffA, asgA, adgA, exvA, hrowA, semA1, semA2, semA3)
    setB = (srcB, dstB, soffB, doffB, asgB, adgB, exvB, hrowB, semB1, semB2, semB3)

    def _issue(i, bufset):
        srcv, dstv, soffv, doffv, asg, adg, exv, hrow, sg, sh, sc = bufset
        base = s * ept + i * CH
        pltpu.sync_copy(src_hbm.at[pl.ds(base, CH)], srcv)
        pltpu.sync_copy(dst_hbm.at[pl.ds(base, CH)], dstv)
        for j in range(CH // 16):
            sl = pl.ds(j * 16, 16)
            soffv[sl] = srcv[sl] + koff
            doffv[sl] = dstv[sl] + koff
        return (pltpu.async_copy(as_hbm.at[soffv], asg, sg),
                pltpu.async_copy(ad_hbm.at[doffv], adg, sg),
                pltpu.async_copy(h_hbm.at[soffv], hrow, sh))

    def _process(gather_cps, bufset):
        srcv, dstv, soffv, doffv, asg, adg, exv, hrow, sg, sh, sc = bufset
        gather_cps[0].wait()
        gather_cps[1].wait()
        for j in range(CH // 16):
            sl = pl.ds(j * 16, 16)
            av = asg[sl]
            dv = adg[sl]
            al = _lrelu(av + dv)
            cc = _lrelu(amax_vec + dv)
            exv[sl] = jnp.exp(al - cc)
        gather_cps[2].wait()

        def erow(e):
            spl = plsc.load_gather(exv, [jnp.full((16,), e, jnp.int32)])
            for f in range(HID // 16):
                fsl = pl.ds(f * 16, 16)
                hrow[e, fsl] = hrow[e, fsl] * spl
        plsc.parallel_loop(0, CH, 1, unroll=4)(erow)

        return (pltpu.async_copy(exv, den_sh.at[dstv], sc, add=True),
                pltpu.async_copy(hrow, acc_sh.at[dstv], sc, add=True))

    def body(jj, _):
        i0 = 2 * jj
        gA = _issue(i0, setA)
        gB = _issue(i0 + 1, setB)
        scA = _process(gA, setA)
        scB = _process(gB, setB)
        scA[0].wait()
        scA[1].wait()
        scB[0].wait()
        scB[1].wait()
        return 0
    lax.fori_loop(0, chunks // 2, body, 0)

    plsc.subcore_barrier()
    for j in range(ROWS_PT // CH):
        r0 = s * ROWS_PT + j * CH
        pltpu.sync_copy(acc_sh.at[pl.ds(r0, CH)], hrowA)
        pltpu.sync_copy(hrowA, acc_hbm.at[pl.ds(koff + r0, CH)])
        pltpu.sync_copy(den_sh.at[pl.ds(r0, CH)], exvA)
        pltpu.sync_copy(exvA, den_hbm.at[pl.ds(koff + r0, CH)])


def _make_edge_call(epad):
    mesh = plsc.VectorSubcoreMesh(core_axis_name="c", subcore_axis_name="s")
    return pl.kernel(
        functools.partial(_edge_body, epad),
        out_type=(jax.ShapeDtypeStruct((2 * NP, HID), _F32),
                  jax.ShapeDtypeStruct((2 * NP,), _F32)),
        mesh=mesh,
        compiler_params=pltpu.CompilerParams(needs_layout_passes=False),
        scratch_types=(
            [pltpu.VMEM((CH,), jnp.int32)] * 4 +
            [pltpu.VMEM((CH,), _F32)] * 3 +
            [pltpu.VMEM((CH, HID), _F32)] +
            [pltpu.VMEM((CH,), jnp.int32)] * 4 +
            [pltpu.VMEM((CH,), _F32)] * 3 +
            [pltpu.VMEM((CH, HID), _F32)] +
            [pltpu.VMEM((16,), _F32)] +
            [pltpu.VMEM_SHARED((NP, HID), _F32),
             pltpu.VMEM_SHARED((NP,), _F32),
             pltpu.SemaphoreType.DMA,
             pltpu.SemaphoreType.DMA,
             pltpu.SemaphoreType.DMA,
             pltpu.SemaphoreType.DMA,
             pltpu.SemaphoreType.DMA,
             pltpu.SemaphoreType.DMA]
        ),
    )


def kernel(metadata, waveform_features, edge_index, batch,
           W_meta, b_meta, W_wave, b_wave, W_comb, b_comb, bnc_g, bnc_b,
           Wg0, as0, ad0, bg0, bn0_g, bn0_b,
           Wg1, as1, ad1, bg1, bn1_g, bn1_b,
           Wg2, as2, ad2, bg2, bn2_g, bn2_b,
           Wl1, bl1, Wl2, bl2, Wo1, bo1, Wo2, bo2):
    loop = jnp.arange(N, dtype=edge_index.dtype)
    src = jnp.concatenate([edge_index[0], loop])
    dst = jnp.concatenate([edge_index[1], loop])
    e2 = src.shape[0]
    epad = ((e2 + NTILES * CH - 1) // (NTILES * CH)) * (NTILES * CH)
    pad = jnp.full((epad - e2,), NP - 1, dtype=src.dtype)
    src_p = jnp.concatenate([src, pad])
    dst_p = jnp.concatenate([dst, pad])

    edge_call = _make_edge_call(epad)

    h, asf, adf, amax = _enc_call(
        metadata, waveform_features, W_meta, b_meta, W_wave, b_wave,
        W_comb[:HID], W_comb[HID:], b_comb, bnc_g, bnc_b, Wg0, as0, ad0)

    acc0, den0 = edge_call(src_p, dst_p, asf, adf, amax, h)
    x1, h, asf, adf, amax = _post_call_nores(
        acc0, den0, bg0, bn0_g, bn0_b, Wg1, as1, ad1)

    acc1, den1 = edge_call(src_p, dst_p, asf, adf, amax, h)
    x2, h, asf, adf, amax = _post_call_res(
        acc1, den1, bg1, bn1_g, bn1_b, x1, Wg2, as2, ad2)

    acc2, den2 = edge_call(src_p, dst_p, asf, adf, amax, h)
    lat, lon = _final_call(
        acc2, den2, bg2, bn2_g, bn2_b, x2, batch,
        Wl1, bl1, Wl2, bl2, Wo1, bo1, Wo2, bo2)
    return lat, lon
